# baseline (speedup 1.0000x reference)
import functools

import jax
import jax.numpy as jnp
from jax import lax
from jax.experimental import pallas as pl
from jax.experimental.pallas import tpu as pltpu
from jax.experimental.pallas import tpu_sc as plsc

_N = 16384
_C = 1000
_NC = 2
_NS = 16
_NW = _NC * _NS
_RPW = _N // _NW
_RCH = 16
_NCH = _RPW // _RCH
_K = max(1, int(_N * 0.7))

@functools.cache
def _build_sc_stage():
    mesh = plsc.VectorSubcoreMesh(core_axis_name="c", subcore_axis_name="s",
                                  num_cores=_NC, num_subcores=_NS)
    return functools.partial(
        pl.kernel,
        out_type=(jax.ShapeDtypeStruct((_N,), jnp.float32),
                  jax.ShapeDtypeStruct((_N,), jnp.float32),
                  jax.ShapeDtypeStruct((_N,), jnp.float32)),
        mesh=mesh,
        compiler_params=pltpu.CompilerParams(needs_layout_passes=False),
        scratch_types=[
            pltpu.VMEM((_RCH * _C,), jnp.float32),
            pltpu.VMEM((_RPW,), jnp.int32),
            pltpu.VMEM((_RPW,), jnp.float32),
            pltpu.VMEM((_RPW,), jnp.float32),
            pltpu.VMEM((_RPW,), jnp.float32),
        ],
    )(_sc_body)


def _sc_body(x_hbm, tgt_hbm, m_hbm, s_hbm, g_hbm, buf, tgt_v, m_v, s_v, g_v):
    wid = lax.axis_index("s") * _NC + lax.axis_index("c")
    row0 = wid * _RPW
    pltpu.sync_copy(tgt_hbm.at[pl.ds(row0, _RPW)], tgt_v)
    lanes = lax.broadcasted_iota(jnp.int32, (16,), 0)
    base_idx = lanes * _C

    def chunk_body(k, _):
        pltpu.sync_copy(x_hbm.at[pl.ds((row0 + k * _RCH) * _C, _RCH * _C)], buf)

        def max_body(c, mx):
            return jnp.maximum(mx, plsc.load_gather(buf, [base_idx + c]))

        m_vec = lax.fori_loop(0, _C, max_body,
                              jnp.full((16,), -jnp.inf, jnp.float32))

        def sum_body(c, acc):
            xv = plsc.load_gather(buf, [base_idx + c])
            return acc + jnp.exp(xv - m_vec)

        s_vec = lax.fori_loop(0, _C, sum_body, jnp.zeros((16,), jnp.float32))

        tcol = tgt_v[pl.ds(k * _RCH, _RCH)]
        g_vec = plsc.load_gather(buf, [base_idx + tcol])

        m_v[pl.ds(k * _RCH, _RCH)] = m_vec
        s_v[pl.ds(k * _RCH, _RCH)] = s_vec
        g_v[pl.ds(k * _RCH, _RCH)] = g_vec
        return 0

    lax.fori_loop(0, _NCH, chunk_body, 0)
    pltpu.sync_copy(m_v, m_hbm.at[pl.ds(row0, _RPW)])
    pltpu.sync_copy(s_v, s_hbm.at[pl.ds(row0, _RPW)])
    pltpu.sync_copy(g_v, g_hbm.at[pl.ds(row0, _RPW)])


def _tc_body(m_ref, s_ref, g_ref, loss_ref, mask_ref):
    m = m_ref[...]
    s = s_ref[...]
    g = g_ref[...]
    ce = m + jnp.log(s) - g
    pt = jnp.exp(-ce)
    focal = 0.25 * (1.0 - pt) ** 2 * ce
    u = lax.bitcast_convert_type(focal, jnp.int32)

    def bit_body(i, th):
        cand = th | (jnp.int32(1) << (30 - i))
        cnt = jnp.sum((u >= cand).astype(jnp.int32))
        return lax.select(cnt >= _K, cand, th)

    th = lax.fori_loop(0, 31, bit_body, jnp.int32(0))
    thf = lax.bitcast_convert_type(th, jnp.float32)
    mask = focal >= thf
    maskf = mask.astype(jnp.float32)
    ksum = jnp.sum(jnp.where(mask, focal, 0.0))
    kcnt = jnp.sum(maskf)
    loss_ref[0, 0] = ksum / kcnt
    mask_ref[...] = maskf


def _tc_stage(m, s, g):
    return pl.pallas_call(
        _tc_body,
        out_shape=(jax.ShapeDtypeStruct((1, 1), jnp.float32),
                   jax.ShapeDtypeStruct((128, 128), jnp.float32)),
        in_specs=[pl.BlockSpec(memory_space=pltpu.VMEM)] * 3,
        out_specs=(pl.BlockSpec(memory_space=pltpu.SMEM),
                   pl.BlockSpec(memory_space=pltpu.VMEM)),
    )(m, s, g)


def kernel(inputs, targets):
    m, s, g = _build_sc_stage()(inputs.reshape(-1), targets)
    loss, maskf = _tc_stage(m.reshape(128, 128), s.reshape(128, 128),
                            g.reshape(128, 128))
    return (loss.reshape(()), maskf.reshape(-1).astype(bool))


# speedup vs baseline: 2.2151x; 2.2151x over previous
import functools

import jax
import jax.numpy as jnp
from jax import lax
from jax.experimental import pallas as pl
from jax.experimental.pallas import tpu as pltpu
from jax.experimental.pallas import tpu_sc as plsc

_N = 16384
_C = 1000
_NC = 2
_NS = 16
_NW = _NC * _NS
_RPW = _N // _NW
_RCH = 16
_NCH = _RPW // _RCH
_K = max(1, int(_N * 0.7))

@functools.cache
def _build_sc_stage():
    mesh = plsc.VectorSubcoreMesh(core_axis_name="c", subcore_axis_name="s",
                                  num_cores=_NC, num_subcores=_NS)
    return functools.partial(
        pl.kernel,
        out_type=(jax.ShapeDtypeStruct((_N,), jnp.float32),
                  jax.ShapeDtypeStruct((_N,), jnp.float32),
                  jax.ShapeDtypeStruct((_N,), jnp.float32)),
        mesh=mesh,
        compiler_params=pltpu.CompilerParams(needs_layout_passes=False),
        scratch_types=[
            pltpu.VMEM((_RCH * _C,), jnp.float32),
            pltpu.VMEM((_RCH * _C,), jnp.float32),
            pltpu.VMEM((_RPW,), jnp.int32),
            pltpu.VMEM((_RPW,), jnp.float32),
            pltpu.VMEM((_RPW,), jnp.float32),
            pltpu.VMEM((_RPW,), jnp.float32),
            pltpu.SemaphoreType.DMA,
            pltpu.SemaphoreType.DMA,
        ],
    )(_sc_body)


def _tree_reduce(op, xs):
    while len(xs) > 1:
        xs = [op(xs[2 * i], xs[2 * i + 1]) for i in range(len(xs) // 2)]
    return xs[0]


_UNROLL = 8


def _sc_body(x_hbm, tgt_hbm, m_hbm, s_hbm, g_hbm, buf0, buf1, tgt_v,
             m_v, s_v, g_v, sem0, sem1):
    wid = lax.axis_index("s") * _NC + lax.axis_index("c")
    row0 = wid * _RPW
    pltpu.sync_copy(tgt_hbm.at[pl.ds(row0, _RPW)], tgt_v)
    lanes = lax.broadcasted_iota(jnp.int32, (16,), 0)
    base_idx = lanes * _C

    def chunk_slice(k):
        return x_hbm.at[pl.ds((row0 + k * _RCH) * _C, _RCH * _C)]

    def compute(buf, k):
        mx_init = tuple(plsc.load_gather(buf, [base_idx + u])
                        for u in range(_UNROLL))

        def max_body(c, mx):
            xs = [plsc.load_gather(buf, [base_idx + (c + u)])
                  for u in range(_UNROLL)]
            return tuple(jnp.maximum(m, x) for m, x in zip(mx, xs))

        mxs = plsc.parallel_loop(_UNROLL, _C, _UNROLL, carry=mx_init)(max_body)
        m_vec = _tree_reduce(jnp.maximum, list(mxs))

        def sum_body(c, accs):
            a0, a1 = accs
            es = [jnp.exp(plsc.load_gather(buf, [base_idx + (c + u)]) - m_vec)
                  for u in range(_UNROLL)]
            h = _UNROLL // 2
            a0 = a0 + _tree_reduce(jnp.add, es[:h])
            a1 = a1 + _tree_reduce(jnp.add, es[h:])
            return (a0, a1)

        zero = jnp.zeros((16,), jnp.float32)
        a0, a1 = plsc.parallel_loop(0, _C, _UNROLL, carry=(zero, zero))(sum_body)
        s_vec = a0 + a1

        tcol = tgt_v[pl.ds(k * _RCH, _RCH)]
        g_vec = plsc.load_gather(buf, [base_idx + tcol])

        m_v[pl.ds(k * _RCH, _RCH)] = m_vec
        s_v[pl.ds(k * _RCH, _RCH)] = s_vec
        g_v[pl.ds(k * _RCH, _RCH)] = g_vec

    pltpu.async_copy(chunk_slice(0), buf0, sem0)

    def pair_body(k2, _):
        c0 = 2 * k2
        pltpu.make_async_copy(chunk_slice(c0), buf0, sem0).wait()
        pltpu.async_copy(chunk_slice(c0 + 1), buf1, sem1)
        compute(buf0, c0)
        pltpu.make_async_copy(chunk_slice(c0 + 1), buf1, sem1).wait()

        @pl.when(k2 + 1 < _NCH // 2)
        def _():
            pltpu.async_copy(chunk_slice(c0 + 2), buf0, sem0)

        compute(buf1, c0 + 1)
        return 0

    lax.fori_loop(0, _NCH // 2, pair_body, 0)
    pltpu.sync_copy(m_v, m_hbm.at[pl.ds(row0, _RPW)])
    pltpu.sync_copy(s_v, s_hbm.at[pl.ds(row0, _RPW)])
    pltpu.sync_copy(g_v, g_hbm.at[pl.ds(row0, _RPW)])


def _tc_body(m_ref, s_ref, g_ref, loss_ref, mask_ref):
    m = m_ref[...]
    s = s_ref[...]
    g = g_ref[...]
    ce = m + jnp.log(s) - g
    pt = jnp.exp(-ce)
    focal = 0.25 * (1.0 - pt) ** 2 * ce
    u = lax.bitcast_convert_type(focal, jnp.int32)

    def bit_body(i, th):
        cand = th | (jnp.int32(1) << (30 - i))
        cnt = jnp.sum((u >= cand).astype(jnp.int32))
        return lax.select(cnt >= _K, cand, th)

    th = lax.fori_loop(0, 31, bit_body, jnp.int32(0))
    thf = lax.bitcast_convert_type(th, jnp.float32)
    mask = focal >= thf
    maskf = mask.astype(jnp.float32)
    ksum = jnp.sum(jnp.where(mask, focal, 0.0))
    kcnt = jnp.sum(maskf)
    loss_ref[0, 0] = ksum / kcnt
    mask_ref[...] = maskf


def _tc_stage(m, s, g):
    return pl.pallas_call(
        _tc_body,
        out_shape=(jax.ShapeDtypeStruct((1, 1), jnp.float32),
                   jax.ShapeDtypeStruct((128, 128), jnp.float32)),
        in_specs=[pl.BlockSpec(memory_space=pltpu.VMEM)] * 3,
        out_specs=(pl.BlockSpec(memory_space=pltpu.SMEM),
                   pl.BlockSpec(memory_space=pltpu.VMEM)),
    )(m, s, g)


def kernel(inputs, targets):
    m, s, g = _build_sc_stage()(inputs.reshape(-1), targets)
    loss, maskf = _tc_stage(m.reshape(128, 128), s.reshape(128, 128),
                            g.reshape(128, 128))
    return (loss.reshape(()), maskf.reshape(-1).astype(bool))
